# Initial kernel scaffold; baseline (speedup 1.0000x reference)
import jax
import jax.numpy as jnp
from jax.experimental import pallas as pl


def kernel(inputs, codebook):
    raise NotImplementedError("write your pallas kernel here")


# speedup vs baseline: 1.2481x; 1.2481x over previous
import functools

import jax
import jax.numpy as jnp
from jax import lax
from jax.experimental import pallas as pl
from jax.experimental.pallas import tpu as pltpu
from jax.experimental.pallas import tpu_sc as plsc

NUM_EMBED = 8192
DIM = 32
N_TOKENS = 16384
BLK = 256
N_BLOCKS = N_TOKENS // BLK

SC_CORES = 2
SC_SUBCORES = 16
SC_WORKERS = SC_CORES * SC_SUBCORES
ROWS_PER_WORKER = N_TOKENS // SC_WORKERS
GATHER_CHUNK = 128
CHUNKS = ROWS_PER_WORKER // GATHER_CHUNK


def _tc_body(x_ref, x2_ref, cb_ref, c2_ref,
             idx_ref, perp_ref, closs_ref, comm_ref, cbt_ref,
             counts_ref, sqerr_ref):
    i = pl.program_id(0)
    x = x_ref[...]
    cb = cb_ref[...]

    mm = lax.dot_general(x, cb, (((1,), (0,)), ((), ())),
                         preferred_element_type=jnp.float32)
    d = x2_ref[...] - 2.0 * mm + c2_ref[...]

    m = jnp.min(d, axis=1)
    idx = jnp.argmin(d, axis=1).astype(jnp.int32)
    idx_ref[...] = idx.reshape(1, 1, BLK)

    onehot = (idx[:, None]
              == lax.broadcasted_iota(jnp.int32, (BLK, NUM_EMBED), 1))
    blk_counts = jnp.sum(onehot.astype(jnp.float32), axis=0, keepdims=True)
    blk_err = jnp.sum(m)

    @pl.when(i == 0)
    def _init():
        counts_ref[...] = blk_counts
        sqerr_ref[0] = blk_err
        cbt_ref[...] = cb.T

    @pl.when(i > 0)
    def _acc():
        counts_ref[...] += blk_counts
        sqerr_ref[0] += blk_err

    @pl.when(i == N_BLOCKS - 1)
    def _finish():
        avg = counts_ref[...] * (1.0 / N_TOKENS)
        ent = jnp.sum(avg * jnp.log(avg + 1e-10))
        perp_ref[...] = jnp.exp(-ent).reshape(1, 1)
        closs = sqerr_ref[0] * (1.0 / (N_TOKENS * DIM))
        closs_ref[...] = closs.reshape(1, 1)
        comm_ref[...] = (0.25 * closs).reshape(1, 1)


def _tc_stage(flat, x2, codebook, c2):
    return pl.pallas_call(
        _tc_body,
        grid=(N_BLOCKS,),
        in_specs=[
            pl.BlockSpec((BLK, DIM), lambda i: (i, 0)),
            pl.BlockSpec((BLK, 1), lambda i: (i, 0)),
            pl.BlockSpec((DIM, NUM_EMBED), lambda i: (0, 0)),
            pl.BlockSpec((1, NUM_EMBED), lambda i: (0, 0)),
        ],
        out_specs=[
            pl.BlockSpec((1, 1, BLK), lambda i: (i, 0, 0)),
            pl.BlockSpec((1, 1), lambda i: (0, 0)),
            pl.BlockSpec((1, 1), lambda i: (0, 0)),
            pl.BlockSpec((1, 1), lambda i: (0, 0)),
            pl.BlockSpec((NUM_EMBED, DIM), lambda i: (0, 0)),
        ],
        out_shape=[
            jax.ShapeDtypeStruct((N_BLOCKS, 1, BLK), jnp.int32),
            jax.ShapeDtypeStruct((1, 1), jnp.float32),
            jax.ShapeDtypeStruct((1, 1), jnp.float32),
            jax.ShapeDtypeStruct((1, 1), jnp.float32),
            jax.ShapeDtypeStruct((NUM_EMBED, DIM), jnp.float32),
        ],
        scratch_shapes=[
            pltpu.VMEM((1, NUM_EMBED), jnp.float32),
            pltpu.SMEM((1,), jnp.float32),
        ],
    )(flat, x2, codebook, c2)


def _sc_gather_body(cbt_hbm, idx_hbm, out_hbm, idx_v, rows_v, sem):
    wid = lax.axis_index("s") * SC_CORES + lax.axis_index("c")
    base = wid * CHUNKS
    pltpu.sync_copy(idx_hbm.at[pl.ds(base, CHUNKS)], idx_v)
    copies = []
    for k in range(CHUNKS):
        copies.append(
            pltpu.async_copy(cbt_hbm.at[idx_v.at[k]], rows_v.at[k], sem))
    for cp in copies:
        cp.wait()
    pltpu.sync_copy(rows_v, out_hbm.at[pl.ds(base, CHUNKS)])


def _sc_gather(cbt, idx2):
    run = pl.kernel(
        _sc_gather_body,
        out_type=jax.ShapeDtypeStruct((N_TOKENS // GATHER_CHUNK,
                                       GATHER_CHUNK, DIM), jnp.float32),
        mesh=plsc.VectorSubcoreMesh(core_axis_name="c", subcore_axis_name="s"),
        scratch_types=[
            pltpu.VMEM((CHUNKS, GATHER_CHUNK), jnp.int32),
            pltpu.VMEM((CHUNKS, GATHER_CHUNK, DIM), jnp.float32),
            pltpu.SemaphoreType.DMA,
        ],
        compiler_params=pltpu.CompilerParams(use_tc_tiling_on_sc=False),
    )
    return run(cbt, idx2)


def kernel(inputs, codebook):
    flat = inputs.reshape(-1, DIM)
    x2 = jnp.sum(flat ** 2, axis=1, keepdims=True)
    c2 = jnp.sum(codebook ** 2, axis=0, keepdims=True)
    idx3, perp, closs, comm, cbt = _tc_stage(flat, x2, codebook, c2)
    idx2 = idx3.reshape(N_TOKENS // GATHER_CHUNK, GATHER_CHUNK)
    quant = _sc_gather(cbt, idx2)
    ste = quant.reshape(inputs.shape)
    return (ste, perp.reshape(()), closs.reshape(()), comm.reshape(()))


# speedup vs baseline: 1.4252x; 1.1420x over previous
import functools

import jax
import jax.numpy as jnp
from jax import lax
from jax.experimental import pallas as pl
from jax.experimental.pallas import tpu as pltpu
from jax.experimental.pallas import tpu_sc as plsc

NUM_EMBED = 8192
DIM = 32
N_TOKENS = 16384
BLK = 256
N_BLOCKS = N_TOKENS // BLK

SC_CORES = 2
SC_SUBCORES = 16
SC_WORKERS = SC_CORES * SC_SUBCORES
ROWS_PER_WORKER = N_TOKENS // SC_WORKERS
GATHER_CHUNK = 128
CHUNKS = ROWS_PER_WORKER // GATHER_CHUNK


def _tc_body(x_ref, x2_ref, cb_ref, c2_ref,
             idx_ref, perp_ref, closs_ref, comm_ref, cbt_ref,
             counts_ref, sqerr_ref):
    i = pl.program_id(0)
    x = x_ref[...]
    cb = cb_ref[...]

    mm = lax.dot_general(x * -2.0, cb, (((1,), (0,)), ((), ())),
                         preferred_element_type=jnp.float32)
    x2 = x2_ref[...]
    c2 = c2_ref[...]

    LW = 128
    lane = lax.broadcasted_iota(jnp.int32, (BLK, LW), 1)
    val = (x2 + mm[:, 0:LW]) + c2[:, 0:LW]
    idx = lane
    for b in range(1, NUM_EMBED // LW):
        lo, hi = b * LW, (b + 1) * LW
        db = (x2 + mm[:, lo:hi]) + c2[:, lo:hi]
        cmp = db < val
        val = jnp.where(cmp, db, val)
        idx = jnp.where(cmp, lane + (b * LW), idx)
    m = jnp.min(val, axis=1)
    big = jnp.int32(jnp.iinfo(jnp.int32).max)
    idx = jnp.min(jnp.where(val == m[:, None], idx, big), axis=1)
    idx_ref[...] = idx.reshape(1, 1, BLK)

    onehot = (idx[:, None]
              == lax.broadcasted_iota(jnp.int32, (BLK, NUM_EMBED), 1))
    blk_counts = jnp.sum(onehot.astype(jnp.float32), axis=0, keepdims=True)
    blk_err = jnp.sum(m)

    @pl.when(i == 0)
    def _init():
        counts_ref[...] = blk_counts
        sqerr_ref[0] = blk_err
        cbt_ref[...] = cb.T

    @pl.when(i > 0)
    def _acc():
        counts_ref[...] += blk_counts
        sqerr_ref[0] += blk_err

    @pl.when(i == N_BLOCKS - 1)
    def _finish():
        avg = counts_ref[...] * (1.0 / N_TOKENS)
        ent = jnp.sum(avg * jnp.log(avg + 1e-10))
        perp_ref[...] = jnp.exp(-ent).reshape(1, 1)
        closs = sqerr_ref[0] * (1.0 / (N_TOKENS * DIM))
        closs_ref[...] = closs.reshape(1, 1)
        comm_ref[...] = (0.25 * closs).reshape(1, 1)


def _tc_stage(flat, x2, codebook, c2):
    return pl.pallas_call(
        _tc_body,
        grid=(N_BLOCKS,),
        in_specs=[
            pl.BlockSpec((BLK, DIM), lambda i: (i, 0)),
            pl.BlockSpec((BLK, 1), lambda i: (i, 0)),
            pl.BlockSpec((DIM, NUM_EMBED), lambda i: (0, 0)),
            pl.BlockSpec((1, NUM_EMBED), lambda i: (0, 0)),
        ],
        out_specs=[
            pl.BlockSpec((1, 1, BLK), lambda i: (i, 0, 0)),
            pl.BlockSpec((1, 1), lambda i: (0, 0)),
            pl.BlockSpec((1, 1), lambda i: (0, 0)),
            pl.BlockSpec((1, 1), lambda i: (0, 0)),
            pl.BlockSpec((NUM_EMBED, DIM), lambda i: (0, 0)),
        ],
        out_shape=[
            jax.ShapeDtypeStruct((N_BLOCKS, 1, BLK), jnp.int32),
            jax.ShapeDtypeStruct((1, 1), jnp.float32),
            jax.ShapeDtypeStruct((1, 1), jnp.float32),
            jax.ShapeDtypeStruct((1, 1), jnp.float32),
            jax.ShapeDtypeStruct((NUM_EMBED, DIM), jnp.float32),
        ],
        scratch_shapes=[
            pltpu.VMEM((1, NUM_EMBED), jnp.float32),
            pltpu.SMEM((1,), jnp.float32),
        ],
    )(flat, x2, codebook, c2)


def _sc_gather_body(cbt_hbm, idx_hbm, out_hbm, idx_v, rows_v, sem):
    wid = lax.axis_index("s") * SC_CORES + lax.axis_index("c")
    base = wid * CHUNKS
    pltpu.sync_copy(idx_hbm.at[pl.ds(base, CHUNKS)], idx_v)
    copies = []
    for k in range(CHUNKS):
        copies.append(
            pltpu.async_copy(cbt_hbm.at[idx_v.at[k]], rows_v.at[k], sem))
    for cp in copies:
        cp.wait()
    pltpu.sync_copy(rows_v, out_hbm.at[pl.ds(base, CHUNKS)])


def _sc_gather(cbt, idx2):
    run = pl.kernel(
        _sc_gather_body,
        out_type=jax.ShapeDtypeStruct((N_TOKENS // GATHER_CHUNK,
                                       GATHER_CHUNK, DIM), jnp.float32),
        mesh=plsc.VectorSubcoreMesh(core_axis_name="c", subcore_axis_name="s"),
        scratch_types=[
            pltpu.VMEM((CHUNKS, GATHER_CHUNK), jnp.int32),
            pltpu.VMEM((CHUNKS, GATHER_CHUNK, DIM), jnp.float32),
            pltpu.SemaphoreType.DMA,
        ],
        compiler_params=pltpu.CompilerParams(use_tc_tiling_on_sc=False),
    )
    return run(cbt, idx2)


def kernel(inputs, codebook):
    flat = inputs.reshape(-1, DIM)
    x2 = jnp.sum(flat ** 2, axis=1, keepdims=True)
    c2 = jnp.sum(codebook ** 2, axis=0, keepdims=True)
    idx3, perp, closs, comm, cbt = _tc_stage(flat, x2, codebook, c2)
    idx2 = idx3.reshape(N_TOKENS // GATHER_CHUNK, GATHER_CHUNK)
    quant = _sc_gather(cbt, idx2)
    ste = quant.reshape(inputs.shape)
    return (ste, perp.reshape(()), closs.reshape(()), comm.reshape(()))


# speedup vs baseline: 1.6708x; 1.1723x over previous
import jax
import jax.numpy as jnp
from jax import lax
from jax.experimental import pallas as pl
from jax.experimental.pallas import tpu as pltpu
from jax.experimental.pallas import tpu_sc as plsc

NUM_EMBED = 8192
DIM = 32
N_TOKENS = 16384
BLK = 256
N_BLOCKS = N_TOKENS // BLK

SC_CORES = 2
SC_SUBCORES = 16
SC_WORKERS = SC_CORES * SC_SUBCORES
ROWS_PER_WORKER = N_TOKENS // SC_WORKERS
GATHER_CHUNK = 128
CHUNKS = ROWS_PER_WORKER // GATHER_CHUNK
PAD = 8
BINS_PER_SUB = NUM_EMBED // SC_SUBCORES


def _tc_body(x_ref, x2_ref, cb_ref, c2_ref,
             idx_ref, closs_ref, comm_ref, cbt_ref, ones_ref, zeros_ref,
             sqerr_ref):
    i = pl.program_id(0)
    x = x_ref[...]
    cb = cb_ref[...]

    mm = lax.dot_general(x * -2.0, cb, (((1,), (0,)), ((), ())),
                         preferred_element_type=jnp.float32)
    d = (x2_ref[...] + mm) + c2_ref[...]

    m = jnp.min(d, axis=1)
    idx = jnp.argmin(d, axis=1).astype(jnp.int32)
    idx_ref[...] = idx.reshape(1, 1, BLK)
    blk_err = jnp.sum(m)

    @pl.when(i == 0)
    def _init():
        sqerr_ref[0] = blk_err
        cbt_ref[...] = cb.T
        ones_ref[...] = jnp.ones((GATHER_CHUNK, PAD), jnp.float32)
        zeros_ref[...] = jnp.zeros((BINS_PER_SUB, PAD), jnp.float32)

    @pl.when(i > 0)
    def _acc():
        sqerr_ref[0] += blk_err

    @pl.when(i == N_BLOCKS - 1)
    def _finish():
        closs = sqerr_ref[0] * (1.0 / (N_TOKENS * DIM))
        closs_ref[...] = closs.reshape(1, 1)
        comm_ref[...] = (0.25 * closs).reshape(1, 1)


def _tc_stage(flat, x2, codebook, c2):
    return pl.pallas_call(
        _tc_body,
        grid=(N_BLOCKS,),
        in_specs=[
            pl.BlockSpec((BLK, DIM), lambda i: (i, 0)),
            pl.BlockSpec((BLK, 1), lambda i: (i, 0)),
            pl.BlockSpec((DIM, NUM_EMBED), lambda i: (0, 0)),
            pl.BlockSpec((1, NUM_EMBED), lambda i: (0, 0)),
        ],
        out_specs=[
            pl.BlockSpec((1, 1, BLK), lambda i: (i, 0, 0)),
            pl.BlockSpec((1, 1), lambda i: (0, 0)),
            pl.BlockSpec((1, 1), lambda i: (0, 0)),
            pl.BlockSpec((NUM_EMBED, DIM), lambda i: (0, 0)),
            pl.BlockSpec((GATHER_CHUNK, PAD), lambda i: (0, 0)),
            pl.BlockSpec((BINS_PER_SUB, PAD), lambda i: (0, 0)),
        ],
        out_shape=[
            jax.ShapeDtypeStruct((N_BLOCKS, 1, BLK), jnp.int32),
            jax.ShapeDtypeStruct((1, 1), jnp.float32),
            jax.ShapeDtypeStruct((1, 1), jnp.float32),
            jax.ShapeDtypeStruct((NUM_EMBED, DIM), jnp.float32),
            jax.ShapeDtypeStruct((GATHER_CHUNK, PAD), jnp.float32),
            jax.ShapeDtypeStruct((BINS_PER_SUB, PAD), jnp.float32),
        ],
        scratch_shapes=[
            pltpu.SMEM((1,), jnp.float32),
        ],
    )(flat, x2, codebook, c2)


def _sc_body(cbt_hbm, idx_hbm, ones_hbm, zeros_hbm, q_out, cnt_out,
             idx_v, rows_v, ones_v, zeros_v, cnt_v, cnt_sh, sem):
    cid = lax.axis_index("c")
    sid = lax.axis_index("s")
    wid = sid * SC_CORES + cid
    base = wid * CHUNKS
    pltpu.sync_copy(idx_hbm.at[pl.ds(base, CHUNKS)], idx_v)
    copies = [
        pltpu.async_copy(cbt_hbm.at[idx_v.at[k]], rows_v.at[k], sem)
        for k in range(CHUNKS)
    ]
    pltpu.sync_copy(zeros_hbm, zeros_v)
    pltpu.sync_copy(zeros_v, cnt_sh.at[pl.ds(sid * BINS_PER_SUB,
                                             BINS_PER_SUB)])
    pltpu.sync_copy(ones_hbm, ones_v)
    plsc.subcore_barrier()
    for k in range(CHUNKS):
        pltpu.sync_copy(ones_v, cnt_sh.at[idx_v.at[k]], add=True)
    plsc.subcore_barrier()
    pltpu.sync_copy(cnt_sh.at[pl.ds(sid * BINS_PER_SUB, BINS_PER_SUB)],
                    cnt_v)
    pltpu.sync_copy(cnt_v, cnt_out.at[cid, pl.ds(sid * BINS_PER_SUB,
                                                 BINS_PER_SUB)])
    for cp in copies:
        cp.wait()
    pltpu.sync_copy(rows_v, q_out.at[pl.ds(base, CHUNKS)])


def _sc_stage(cbt, idx2, ones, zeros):
    run = pl.kernel(
        _sc_body,
        out_type=(
            jax.ShapeDtypeStruct((N_TOKENS // GATHER_CHUNK,
                                  GATHER_CHUNK, DIM), jnp.float32),
            jax.ShapeDtypeStruct((SC_CORES, NUM_EMBED, PAD), jnp.float32),
        ),
        mesh=plsc.VectorSubcoreMesh(core_axis_name="c", subcore_axis_name="s"),
        scratch_types=[
            pltpu.VMEM((CHUNKS, GATHER_CHUNK), jnp.int32),
            pltpu.VMEM((CHUNKS, GATHER_CHUNK, DIM), jnp.float32),
            pltpu.VMEM((GATHER_CHUNK, PAD), jnp.float32),
            pltpu.VMEM((BINS_PER_SUB, PAD), jnp.float32),
            pltpu.VMEM((BINS_PER_SUB, PAD), jnp.float32),
            pltpu.VMEM_SHARED((NUM_EMBED, PAD), jnp.float32),
            pltpu.SemaphoreType.DMA,
        ],
        compiler_params=pltpu.CompilerParams(use_tc_tiling_on_sc=False),
    )
    return run(cbt, idx2, ones, zeros)


def _perp_body(cnt_ref, perp_ref):
    c = cnt_ref[0] + cnt_ref[1]
    avg = c * (1.0 / N_TOKENS)
    ent = jnp.sum(avg * jnp.log(avg + 1e-10))
    perp_ref[...] = jnp.exp(-ent * (1.0 / PAD)).reshape(1, 1)


def _perp_stage(cnt):
    return pl.pallas_call(
        _perp_body,
        out_shape=jax.ShapeDtypeStruct((1, 1), jnp.float32),
    )(cnt)


def kernel(inputs, codebook):
    flat = inputs.reshape(-1, DIM)
    x2 = jnp.sum(flat ** 2, axis=1, keepdims=True)
    c2 = jnp.sum(codebook ** 2, axis=0, keepdims=True)
    idx3, closs, comm, cbt, ones, zeros = _tc_stage(flat, x2, codebook, c2)
    idx2 = idx3.reshape(N_TOKENS // GATHER_CHUNK, GATHER_CHUNK)
    quant, cnt = _sc_stage(cbt, idx2, ones, zeros)
    perp = _perp_stage(cnt)
    ste = quant.reshape(inputs.shape)
    return (ste, perp.reshape(()), closs.reshape(()), comm.reshape(()))


# speedup vs baseline: 1.6964x; 1.0153x over previous
import jax
import jax.numpy as jnp
from jax import lax
from jax.experimental import pallas as pl
from jax.experimental.pallas import tpu as pltpu
from jax.experimental.pallas import tpu_sc as plsc

NUM_EMBED = 8192
DIM = 32
N_TOKENS = 16384
BLK = 256
N_BLOCKS = N_TOKENS // BLK

SC_CORES = 2
SC_SUBCORES = 16
SC_WORKERS = SC_CORES * SC_SUBCORES
ROWS_PER_WORKER = N_TOKENS // SC_WORKERS
GATHER_CHUNK = 128
CHUNKS = ROWS_PER_WORKER // GATHER_CHUNK
PAD = 8
BINS_PER_SUB = NUM_EMBED // SC_SUBCORES


def _tc_body(x_ref, x2_ref, cb_ref, c2_ref,
             idx_ref, closs_ref, comm_ref, cbt_ref,
             sqerr_ref):
    i = pl.program_id(0)
    x = x_ref[...]
    cb = cb_ref[...]

    mm = lax.dot_general(x * -2.0, cb, (((1,), (0,)), ((), ())),
                         preferred_element_type=jnp.float32)
    d = (x2_ref[...] + mm) + c2_ref[...]

    m = jnp.min(d, axis=1)
    idx = jnp.argmin(d, axis=1).astype(jnp.int32)
    idx_ref[...] = idx.reshape(1, 1, BLK)
    blk_err = jnp.sum(m)

    lo = i * (NUM_EMBED // N_BLOCKS)
    cbt_ref[...] = lax.transpose(
        cb_ref[:, pl.ds(lo, NUM_EMBED // N_BLOCKS)], (1, 0))

    @pl.when(i == 0)
    def _init():
        sqerr_ref[0] = blk_err

    @pl.when(i > 0)
    def _acc():
        sqerr_ref[0] += blk_err

    @pl.when(i == N_BLOCKS - 1)
    def _finish():
        closs = sqerr_ref[0] * (1.0 / (N_TOKENS * DIM))
        closs_ref[...] = closs.reshape(1, 1)
        comm_ref[...] = (0.25 * closs).reshape(1, 1)


def _tc_stage(flat, x2, codebook, c2):
    return pl.pallas_call(
        _tc_body,
        grid=(N_BLOCKS,),
        in_specs=[
            pl.BlockSpec((BLK, DIM), lambda i: (i, 0)),
            pl.BlockSpec((BLK, 1), lambda i: (i, 0)),
            pl.BlockSpec((DIM, NUM_EMBED), lambda i: (0, 0)),
            pl.BlockSpec((1, NUM_EMBED), lambda i: (0, 0)),
        ],
        out_specs=[
            pl.BlockSpec((1, 1, BLK), lambda i: (i, 0, 0)),
            pl.BlockSpec((1, 1), lambda i: (0, 0)),
            pl.BlockSpec((1, 1), lambda i: (0, 0)),
            pl.BlockSpec((NUM_EMBED // N_BLOCKS, DIM), lambda i: (i, 0)),
        ],
        out_shape=[
            jax.ShapeDtypeStruct((N_BLOCKS, 1, BLK), jnp.int32),
            jax.ShapeDtypeStruct((1, 1), jnp.float32),
            jax.ShapeDtypeStruct((1, 1), jnp.float32),
            jax.ShapeDtypeStruct((NUM_EMBED, DIM), jnp.float32),
        ],
        scratch_shapes=[
            pltpu.SMEM((1,), jnp.float32),
        ],
    )(flat, x2, codebook, c2)


def _sc_body(cbt_hbm, idx_hbm, ones_hbm, zeros_hbm, q_out, cnt_out,
             idx_v, rows_v, ones_v, zeros_v, cnt_v, cnt_sh, sem):
    cid = lax.axis_index("c")
    sid = lax.axis_index("s")
    wid = sid * SC_CORES + cid
    base = wid * CHUNKS
    pltpu.sync_copy(idx_hbm.at[pl.ds(base, CHUNKS)], idx_v)
    copies = [
        pltpu.async_copy(cbt_hbm.at[idx_v.at[k]], rows_v.at[k], sem)
        for k in range(CHUNKS)
    ]
    pltpu.sync_copy(zeros_hbm, zeros_v)
    pltpu.sync_copy(zeros_v, cnt_sh.at[pl.ds(sid * BINS_PER_SUB,
                                             BINS_PER_SUB)])
    pltpu.sync_copy(ones_hbm, ones_v)
    plsc.subcore_barrier()
    for k in range(CHUNKS):
        pltpu.sync_copy(ones_v, cnt_sh.at[idx_v.at[k]], add=True)
    plsc.subcore_barrier()
    pltpu.sync_copy(cnt_sh.at[pl.ds(sid * BINS_PER_SUB, BINS_PER_SUB)],
                    cnt_v)
    pltpu.sync_copy(cnt_v, cnt_out.at[cid, pl.ds(sid * BINS_PER_SUB,
                                                 BINS_PER_SUB)])
    for cp in copies:
        cp.wait()
    pltpu.sync_copy(rows_v, q_out.at[pl.ds(base, CHUNKS)])


def _sc_stage(cbt, idx2, ones, zeros):
    run = pl.kernel(
        _sc_body,
        out_type=(
            jax.ShapeDtypeStruct((N_TOKENS // GATHER_CHUNK,
                                  GATHER_CHUNK, DIM), jnp.float32),
            jax.ShapeDtypeStruct((SC_CORES, NUM_EMBED, PAD), jnp.float32),
        ),
        mesh=plsc.VectorSubcoreMesh(core_axis_name="c", subcore_axis_name="s"),
        scratch_types=[
            pltpu.VMEM((CHUNKS, GATHER_CHUNK), jnp.int32),
            pltpu.VMEM((CHUNKS, GATHER_CHUNK, DIM), jnp.float32),
            pltpu.VMEM((GATHER_CHUNK, PAD), jnp.float32),
            pltpu.VMEM((BINS_PER_SUB, PAD), jnp.float32),
            pltpu.VMEM((BINS_PER_SUB, PAD), jnp.float32),
            pltpu.VMEM_SHARED((NUM_EMBED, PAD), jnp.float32),
            pltpu.SemaphoreType.DMA,
        ],
        compiler_params=pltpu.CompilerParams(use_tc_tiling_on_sc=False),
    )
    return run(cbt, idx2, ones, zeros)


def _perp_body(cnt_ref, perp_ref):
    c = cnt_ref[0] + cnt_ref[1]
    avg = c * (1.0 / N_TOKENS)
    ent = jnp.sum(avg * jnp.log(avg + 1e-10))
    perp_ref[...] = jnp.exp(-ent * (1.0 / PAD)).reshape(1, 1)


def _perp_stage(cnt):
    return pl.pallas_call(
        _perp_body,
        out_shape=jax.ShapeDtypeStruct((1, 1), jnp.float32),
    )(cnt)


def kernel(inputs, codebook):
    flat = inputs.reshape(-1, DIM)
    x2 = jnp.sum(flat ** 2, axis=1, keepdims=True)
    c2 = jnp.sum(codebook ** 2, axis=0, keepdims=True)
    idx3, closs, comm, cbt = _tc_stage(flat, x2, codebook, c2)
    idx2 = idx3.reshape(N_TOKENS // GATHER_CHUNK, GATHER_CHUNK)
    ones = jnp.ones((GATHER_CHUNK, PAD), jnp.float32)
    zeros = jnp.zeros((BINS_PER_SUB, PAD), jnp.float32)
    quant, cnt = _sc_stage(cbt, idx2, ones, zeros)
    perp = _perp_stage(cnt)
    ste = quant.reshape(inputs.shape)
    return (ste, perp.reshape(()), closs.reshape(()), comm.reshape(()))


# speedup vs baseline: 1.8846x; 1.1109x over previous
import jax
import jax.numpy as jnp
from jax import lax
from jax.experimental import pallas as pl
from jax.experimental.pallas import tpu as pltpu
from jax.experimental.pallas import tpu_sc as plsc

NUM_EMBED = 8192
DIM = 32
N_TOKENS = 16384
BLK = 256
N_BLOCKS = N_TOKENS // BLK

SC_CORES = 2
SC_SUBCORES = 16
SC_WORKERS = SC_CORES * SC_SUBCORES
ROWS_PER_WORKER = N_TOKENS // SC_WORKERS
GATHER_CHUNK = 128
CHUNKS = ROWS_PER_WORKER // GATHER_CHUNK
PAD = 8
BINS_PER_SUB = NUM_EMBED // SC_SUBCORES


RB = 64
LW = 128


def _tc_body(x_ref, x2_ref, cb_ref, c2_ref,
             idx_ref, closs_ref, comm_ref, cbt_ref,
             sqerr_ref):
    i = pl.program_id(0)
    x = x_ref[...]
    cb = cb_ref[...]

    mm = lax.dot_general(x * -2.0, cb, (((1,), (0,)), ((), ())),
                         preferred_element_type=jnp.float32)
    x2r = lax.transpose(x2_ref[...], (1, 0))
    c2 = c2_ref[...]

    big = jnp.int32(jnp.iinfo(jnp.int32).max)
    lane = lax.broadcasted_iota(jnp.int32, (RB, LW), 1)
    blk_err = jnp.float32(0.0)
    idx_parts = []
    for r in range(BLK // RB):
        rows = slice(r * RB, (r + 1) * RB)
        x2b = x2r[rows]
        val = (x2b + mm[rows, 0:LW]) + c2[:, 0:LW]
        idx = lane
        for p in range(1, NUM_EMBED // LW):
            lo = p * LW
            db = (x2b + mm[rows, lo:lo + LW]) + c2[:, lo:lo + LW]
            cmp = db < val
            val = jnp.where(cmp, db, val)
            idx = jnp.where(cmp, lane + lo, idx)
        m = jnp.min(val, axis=1)
        gi = jnp.min(jnp.where(val == m[:, None], idx, big), axis=1)
        idx_parts.append(gi)
        blk_err = blk_err + jnp.sum(m)
    idx = jnp.concatenate(idx_parts)
    idx_ref[...] = idx.reshape(1, 1, BLK)

    lo = i * (NUM_EMBED // N_BLOCKS)
    cbt_ref[...] = lax.transpose(
        cb_ref[:, pl.ds(lo, NUM_EMBED // N_BLOCKS)], (1, 0))

    @pl.when(i == 0)
    def _init():
        sqerr_ref[0] = blk_err

    @pl.when(i > 0)
    def _acc():
        sqerr_ref[0] += blk_err

    @pl.when(i == N_BLOCKS - 1)
    def _finish():
        closs = sqerr_ref[0] * (1.0 / (N_TOKENS * DIM))
        closs_ref[...] = closs.reshape(1, 1)
        comm_ref[...] = (0.25 * closs).reshape(1, 1)


def _tc_stage(flat, x2, codebook, c2):
    return pl.pallas_call(
        _tc_body,
        grid=(N_BLOCKS,),
        in_specs=[
            pl.BlockSpec((BLK, DIM), lambda i: (i, 0)),
            pl.BlockSpec((1, BLK), lambda i: (0, i)),
            pl.BlockSpec((DIM, NUM_EMBED), lambda i: (0, 0)),
            pl.BlockSpec((1, NUM_EMBED), lambda i: (0, 0)),
        ],
        out_specs=[
            pl.BlockSpec((1, 1, BLK), lambda i: (i, 0, 0)),
            pl.BlockSpec((1, 1), lambda i: (0, 0)),
            pl.BlockSpec((1, 1), lambda i: (0, 0)),
            pl.BlockSpec((NUM_EMBED // N_BLOCKS, DIM), lambda i: (i, 0)),
        ],
        out_shape=[
            jax.ShapeDtypeStruct((N_BLOCKS, 1, BLK), jnp.int32),
            jax.ShapeDtypeStruct((1, 1), jnp.float32),
            jax.ShapeDtypeStruct((1, 1), jnp.float32),
            jax.ShapeDtypeStruct((NUM_EMBED, DIM), jnp.float32),
        ],
        scratch_shapes=[
            pltpu.SMEM((1,), jnp.float32),
        ],
    )(flat, x2, codebook, c2)


def _sc_body(cbt_hbm, idx_hbm, ones_hbm, zeros_hbm, q_out, cnt_out,
             idx_v, rows_v, ones_v, zeros_v, cnt_v, cnt_sh, sem):
    cid = lax.axis_index("c")
    sid = lax.axis_index("s")
    wid = sid * SC_CORES + cid
    base = wid * CHUNKS
    pltpu.sync_copy(zeros_hbm, zeros_v)
    pltpu.sync_copy(zeros_v, cnt_sh.at[pl.ds(sid * BINS_PER_SUB,
                                             BINS_PER_SUB)])
    pltpu.sync_copy(ones_hbm, ones_v)
    plsc.subcore_barrier()
    pltpu.sync_copy(idx_hbm.at[pl.ds(base, CHUNKS)], idx_v)
    copies = [
        pltpu.async_copy(cbt_hbm.at[idx_v.at[k]], rows_v.at[k], sem)
        for k in range(CHUNKS)
    ]
    for k in range(CHUNKS):
        pltpu.sync_copy(ones_v, cnt_sh.at[idx_v.at[k]], add=True)
    plsc.subcore_barrier()
    pltpu.sync_copy(cnt_sh.at[pl.ds(sid * BINS_PER_SUB, BINS_PER_SUB)],
                    cnt_v)
    pltpu.sync_copy(cnt_v, cnt_out.at[cid, pl.ds(sid * BINS_PER_SUB,
                                                 BINS_PER_SUB)])
    for cp in copies:
        cp.wait()
    pltpu.sync_copy(rows_v, q_out.at[pl.ds(base, CHUNKS)])


def _sc_stage(cbt, idx2, ones, zeros):
    run = pl.kernel(
        _sc_body,
        out_type=(
            jax.ShapeDtypeStruct((N_TOKENS // GATHER_CHUNK,
                                  GATHER_CHUNK, DIM), jnp.float32),
            jax.ShapeDtypeStruct((SC_CORES, NUM_EMBED, PAD), jnp.float32),
        ),
        mesh=plsc.VectorSubcoreMesh(core_axis_name="c", subcore_axis_name="s"),
        scratch_types=[
            pltpu.VMEM((CHUNKS, GATHER_CHUNK), jnp.int32),
            pltpu.VMEM((CHUNKS, GATHER_CHUNK, DIM), jnp.float32),
            pltpu.VMEM((GATHER_CHUNK, PAD), jnp.float32),
            pltpu.VMEM((BINS_PER_SUB, PAD), jnp.float32),
            pltpu.VMEM((BINS_PER_SUB, PAD), jnp.float32),
            pltpu.VMEM_SHARED((NUM_EMBED, PAD), jnp.float32),
            pltpu.SemaphoreType.DMA,
        ],
        compiler_params=pltpu.CompilerParams(use_tc_tiling_on_sc=False),
    )
    return run(cbt, idx2, ones, zeros)


def _perp_body(cnt_ref, perp_ref):
    c = cnt_ref[0] + cnt_ref[1]
    avg = c * (1.0 / N_TOKENS)
    ent = jnp.sum(avg * jnp.log(avg + 1e-10))
    perp_ref[...] = jnp.exp(-ent * (1.0 / PAD)).reshape(1, 1)


def _perp_stage(cnt):
    return pl.pallas_call(
        _perp_body,
        out_shape=jax.ShapeDtypeStruct((1, 1), jnp.float32),
    )(cnt)


def kernel(inputs, codebook):
    flat = inputs.reshape(-1, DIM)
    x2 = jnp.sum(flat ** 2, axis=1).reshape(1, N_TOKENS)
    c2 = jnp.sum(codebook ** 2, axis=0, keepdims=True)
    idx3, closs, comm, cbt = _tc_stage(flat, x2, codebook, c2)
    idx2 = idx3.reshape(N_TOKENS // GATHER_CHUNK, GATHER_CHUNK)
    ones = jnp.ones((GATHER_CHUNK, PAD), jnp.float32)
    zeros = jnp.zeros((BINS_PER_SUB, PAD), jnp.float32)
    quant, cnt = _sc_stage(cbt, idx2, ones, zeros)
    perp = _perp_stage(cnt)
    ste = quant.reshape(inputs.shape)
    return (ste, perp.reshape(()), closs.reshape(()), comm.reshape(()))
